# baseline (speedup 1.0000x reference)
import functools

import jax
import jax.numpy as jnp
from jax import lax
from jax.experimental import pallas as pl
from jax.experimental.pallas import tpu as pltpu
from jax.experimental.pallas import tpu_sc as plsc

V = 1000000
RV = 1000
D = 64
B = 16384
NC = 2
NS = 16
NW = NC * NS
BPW = B // NW
WL = 512
WPW = 61
TAIL = WPW * WL * NW + WL
MCAP = 1216
CCAP = 64
SEG = 4096
DUMP = 2 * B

_iota = lambda: lax.iota(jnp.int32, 16)
_splat = lambda s: jnp.full((16,), 0, jnp.int32) + s

_mesh = plsc.VectorSubcoreMesh(core_axis_name="c", subcore_axis_name="s")


@functools.partial(
    pl.kernel,
    mesh=_mesh,
    compiler_params=pltpu.CompilerParams(
        use_tc_tiling_on_sc=True, needs_layout_passes=False),
    out_type=(
        jax.ShapeDtypeStruct((2 * B + 16, 128), jnp.float32),
        jax.ShapeDtypeStruct((B, 128), jnp.float32),
    ),
    scratch_types=[
        pltpu.VMEM((SEG,), jnp.int32),
        pltpu.VMEM((SEG,), jnp.int32),
        pltpu.VMEM((BPW,), jnp.int32),
        pltpu.VMEM((D, 1024), jnp.float32),
        pltpu.VMEM((MCAP,), jnp.int32),
        pltpu.VMEM((MCAP,), jnp.int32),
        pltpu.VMEM((CCAP,), jnp.int32),
        pltpu.VMEM((CCAP,), jnp.int32),
        pltpu.VMEM((16, 128), jnp.float32),
        pltpu.VMEM((16, 128), jnp.float32),
        pltpu.SemaphoreType.DMA,
        pltpu.SemaphoreType.DMA,
        pltpu.SemaphoreType.DMA,
        pltpu.SemaphoreType.DMA,
        pltpu.SemaphoreType.DMA,
        pltpu.SemaphoreType.DMA,
    ],
)
def _transe_scan(h_hbm, r_hbm, t_hbm, entT, relT, tailT,
                 ht_out, r_out,
                 iseg0, iseg1, ridx, A, mv, mp, clv, clp, stage0, stage1,
                 si0, si1, sw0, sw1, so0, so1):
    wid = lax.axis_index("s") * NC + lax.axis_index("c")
    base = wid * BPW
    is31 = (wid == NW - 1).astype(jnp.int32)
    lo = wid * (WPW * WL)
    hi = lo + WPW * WL + is31 * (WL + D)
    nwin = WPW + is31

    iseg = (iseg0, iseg1)
    sseg = (si0, si1)
    swin = (sw0, sw1)
    sout = (so0, so1)
    stage = (stage0, stage1)

    pltpu.sync_copy(r_hbm.at[pl.ds(base, BPW)], ridx)

    units = [(h_hbm, s, 0) for s in range(B // SEG)] + \
            [(t_hbm, s, B) for s in range(B // SEG)]
    seg_cp = [None] * len(units)
    src0, s0, _ = units[0]
    seg_cp[0] = pltpu.async_copy(src0.at[pl.ds(s0 * SEG, SEG)], iseg[0], sseg[0])
    cnt = _splat(0)
    for u, (src, s, poff) in enumerate(units):
        b = u % 2
        if u + 1 < len(units):
            nsrc, ns, _ = units[u + 1]
            seg_cp[u + 1] = pltpu.async_copy(
                nsrc.at[pl.ds(ns * SEG, SEG)], iseg[(u + 1) % 2], sseg[(u + 1) % 2])
        seg_cp[u].wait()

        def fbody(kb, cnt, _b=b, _s=s, _poff=poff):
            v = iseg[_b][pl.ds(kb * 16, 16)]
            m = (v >= lo) & (v < hi)
            offs = cnt + plsc.cumsum(jnp.where(m, 1, 0)) - 1
            m = m & (offs < MCAP)
            pos = _poff + _s * SEG + kb * 16 + _iota()
            plsc.store_scatter(mv, [offs], v, mask=m)
            plsc.store_scatter(mp, [offs], pos, mask=m)
            return cnt + plsc.all_reduce_population_count(m)
        cnt = lax.fori_loop(0, SEG // 16, fbody, cnt)
    mcnt = cnt

    def win_lane(k):
        return pl.multiple_of((wid * WPW + k) * WL, WL)

    def issue_win(k, half):
        return pltpu.async_copy(
            entT.at[:, pl.ds(win_lane(k), WL)],
            A.at[:, pl.ds(half * WL, WL)], swin[half])

    def drain(sem, dst):
        pltpu.make_async_copy(entT.at[:, pl.ds(0, WL)]
                              if dst.shape == (D, WL) else
                              ht_out.at[pl.ds(0, 16)], dst, sem).wait()

    pltpu.async_copy(stage0, ht_out.at[DUMP + _iota()], so0)
    pltpu.async_copy(stage1, ht_out.at[DUMP + _iota()], so1)

    issue_win(0, 0)
    issue_win(1, 1)

    def bucket(wbase, span, mcnt):
        def bbody(t, ccnt):
            v = mv[pl.ds(t * 16, 16)]
            p = mp[pl.ds(t * 16, 16)]
            m = ((t * 16 + _iota()) < mcnt) & (v >= wbase) & (v < wbase + span)
            offs = ccnt + plsc.cumsum(jnp.where(m, 1, 0)) - 1
            m = m & (offs < CCAP)
            plsc.store_scatter(clv, [offs], v, mask=m)
            plsc.store_scatter(clp, [offs], p, mask=m)
            return ccnt + plsc.all_reduce_population_count(m)
        return lax.fori_loop(0, MCAP // 16, bbody, _splat(0))

    def extract_group(g, lanebase_sub, ccnt, lane_extra):
        gvalid = (g * 16 + _iota()) < ccnt
        lv = clv[pl.ds(g * 16, 16)]
        lp = clp[pl.ds(g * 16, 16)]
        lane = jnp.where(gvalid, lv - lanebase_sub, 0) + lane_extra

        def cbody(c, _):
            val = plsc.load_gather(A, [_splat(c), lane])
            plsc.store_scatter(stage[g % 2], [_iota(), _splat(c)], val)
            return 0
        lax.fori_loop(0, D, cbody, 0)
        pos = jnp.where(gvalid, lp, DUMP + _iota())
        return pltpu.async_copy(stage[g % 2], ht_out.at[pos], sout[g % 2])

    def wbody(i, mcnt):
        for bb in range(2):
            k = 2 * i + bb

            @pl.when(k < nwin)
            def _():
                drain(swin[bb], A.at[:, pl.ds(bb * WL, WL)])
                wbase = (wid * WPW + k) * WL
                ccnt = bucket(wbase, WL, mcnt)
                drain(sout[0], stage0)
                drain(sout[1], stage1)
                hnd = [None] * 4
                for g in range(4):
                    if g >= 2:
                        hnd[g - 2].wait()
                    hnd[g] = extract_group(g, wbase, ccnt, bb * WL)

                @pl.when(k + 2 < nwin)
                def _():
                    issue_win(k + 2, bb)
        return mcnt

    lax.fori_loop(0, (WPW + 2) // 2, wbody, mcnt)

    drain(sout[0], stage0)
    drain(sout[1], stage1)

    @pl.when(is31 == 1)
    def _():
        pltpu.sync_copy(tailT, A.at[:, pl.ds(0, 128)])
        ccnt = bucket(TAIL, D, mcnt)
        h0 = extract_group(0, TAIL, ccnt, 0)
        h1 = extract_group(1, TAIL, ccnt, 0)
        h0.wait()
        h1.wait()

    pltpu.sync_copy(relT, A)
    rhnd = [None] * (BPW // 16)
    for g in range(BPW // 16):
        if g >= 2:
            rhnd[g - 2].wait()
        lane = ridx[pl.ds(g * 16, 16)]

        def cbody(c, _, _lane=lane, _g=g):
            val = plsc.load_gather(A, [_splat(c), _lane])
            plsc.store_scatter(stage[_g % 2], [_iota(), _splat(c)], val)
            return 0
        lax.fori_loop(0, D, cbody, 0)
        rhnd[g] = pltpu.async_copy(
            stage[g % 2], r_out.at[pl.ds(base + g * 16, 16)], sout[g % 2])
    rhnd[-2].wait()
    rhnd[-1].wait()


def kernel(h, r, t, ent_emb, rel_emb):
    relT = jnp.pad(rel_emb.T, ((0, 0), (0, 1024 - RV)))
    tailT = jnp.pad(ent_emb[TAIL:].T, ((0, 0), (0, 128 - (V - TAIL))))
    ht, r_rows = _transe_scan(h, r, t, ent_emb.T, relT, tailT)
    return (ht[:B, :D], ht[B:2 * B, :D], r_rows[:, :D])
